# Initial kernel scaffold; baseline (speedup 1.0000x reference)
import jax
import jax.numpy as jnp
from jax.experimental import pallas as pl


def kernel(x, edge_index, batch, W1, b1, W2, b2):
    raise NotImplementedError("write your pallas kernel here")


# speedup vs baseline: 9.9068x; 9.9068x over previous
import functools

import jax
import jax.numpy as jnp
from jax import lax
from jax.experimental import pallas as pl
from jax.experimental.pallas import tpu as pltpu
from jax.experimental.pallas import tpu_sc as plsc

N = 10000
E = 320000
D = 128
G = 128

NC = 2
NS = 16
NPAD = 10240
RPT = NPAD // NS
EPT = 10240
CH = 128
NCH = EPT // CH
E_PAD = NC * NS * EPT

NB = 2048
NBLK = NPAD // NB

_mesh = plsc.VectorSubcoreMesh(
    core_axis_name="c", subcore_axis_name="s", num_cores=NC, num_subcores=NS)


def _make_sc_degree():
    @functools.partial(
        pl.kernel,
        out_type=jax.ShapeDtypeStruct((NC, NPAD, 16), jnp.float32),
        mesh=_mesh,
        scratch_types=[
            pltpu.VMEM((NCH, CH), jnp.int32),
            pltpu.VMEM((CH, 16), jnp.float32),
            pltpu.VMEM_SHARED((NPAD, 16), jnp.float32),
        ],
    )
    def deg_kernel(dst_hbm, ones_hbm, zeros16_hbm, out_hbm, idx_v, ones_v,
                   acc_sh):
        cid = lax.axis_index("c")
        sid = lax.axis_index("s")
        pltpu.sync_copy(zeros16_hbm, acc_sh.at[pl.ds(sid * RPT, RPT)])
        pltpu.sync_copy(ones_hbm, ones_v)
        pltpu.sync_copy(dst_hbm.at[cid, sid], idx_v)
        plsc.subcore_barrier()

        def chunk(c, carry):
            pltpu.sync_copy(ones_v, acc_sh.at[idx_v.at[c]], add=True)
            return carry

        lax.fori_loop(0, NCH, chunk, 0)
        plsc.subcore_barrier()
        pltpu.sync_copy(acc_sh.at[pl.ds(sid * RPT, RPT)],
                        out_hbm.at[cid, pl.ds(sid * RPT, RPT)])

    return deg_kernel


_sc_degree = _make_sc_degree()


def _make_sc_scatter():
    @functools.partial(
        pl.kernel,
        out_type=jax.ShapeDtypeStruct((NC, NPAD, D), jnp.float32),
        mesh=_mesh,
        scratch_types=[
            pltpu.VMEM((NCH, CH), jnp.int32),
            pltpu.VMEM((NCH, CH), jnp.int32),
            pltpu.VMEM((CH, D), jnp.float32),
            pltpu.VMEM_SHARED((NPAD, D), jnp.float32),
            pltpu.SemaphoreType.DMA,
        ],
    )
    def scatter_kernel(hp_hbm, src_hbm, dst_hbm, zeros_hbm, out_hbm,
                       src_v, dst_v, rows_v, acc_sh, sem):
        cid = lax.axis_index("c")
        sid = lax.axis_index("s")
        row0 = sid * RPT

        @pl.when(cid == 0)
        def _():
            pltpu.sync_copy(hp_hbm.at[pl.ds(row0, RPT)],
                            acc_sh.at[pl.ds(row0, RPT)])

        @pl.when(cid != 0)
        def _():
            pltpu.sync_copy(zeros_hbm, acc_sh.at[pl.ds(row0, RPT)])

        pltpu.sync_copy(src_hbm.at[cid, sid], src_v)
        pltpu.sync_copy(dst_hbm.at[cid, sid], dst_v)
        plsc.subcore_barrier()

        def chunk(c, carry):
            pltpu.async_copy(hp_hbm.at[src_v.at[c]], rows_v, sem).wait()
            pltpu.sync_copy(rows_v, acc_sh.at[dst_v.at[c]], add=True)
            return carry

        lax.fori_loop(0, NCH, chunk, 0)
        plsc.subcore_barrier()
        pltpu.sync_copy(acc_sh.at[pl.ds(row0, RPT)],
                        out_hbm.at[cid, pl.ds(row0, RPT)])

    return scatter_kernel


_sc_scatter = _make_sc_scatter()


def _dinv_from_tab(tab_blk):
    deg = tab_blk[0, :, 0] + tab_blk[1, :, 0] + 1.0
    return lax.rsqrt(deg)[:, None]


def _tc_pre_body(x_ref, w_ref, tab_ref, out_ref):
    h = jnp.dot(x_ref[...], w_ref[...], preferred_element_type=jnp.float32)
    out_ref[...] = h * _dinv_from_tab(tab_ref[...])


def _tc_pre(x_pad, W1, tab):
    return pl.pallas_call(
        _tc_pre_body,
        grid=(NBLK,),
        in_specs=[
            pl.BlockSpec((NB, D), lambda i: (i, 0)),
            pl.BlockSpec((D, D), lambda i: (0, 0)),
            pl.BlockSpec((NC, NB, 16), lambda i: (0, i, 0)),
        ],
        out_specs=pl.BlockSpec((NB, D), lambda i: (i, 0)),
        out_shape=jax.ShapeDtypeStruct((NPAD, D), jnp.float32),
    )(x_pad, W1, tab)


def _tc_mid_body(s_ref, tab_ref, b_ref, w_ref, out_ref):
    dinv = _dinv_from_tab(tab_ref[...])
    s = s_ref[0] + s_ref[1]
    z = jnp.maximum(s * dinv + b_ref[...], 0.0)
    h = jnp.dot(z, w_ref[...], preferred_element_type=jnp.float32)
    out_ref[...] = h * dinv


def _tc_mid(s1, tab, b1, W2):
    return pl.pallas_call(
        _tc_mid_body,
        grid=(NBLK,),
        in_specs=[
            pl.BlockSpec((NC, NB, D), lambda i: (0, i, 0)),
            pl.BlockSpec((NC, NB, 16), lambda i: (0, i, 0)),
            pl.BlockSpec((1, D), lambda i: (0, 0)),
            pl.BlockSpec((D, D), lambda i: (0, 0)),
        ],
        out_specs=pl.BlockSpec((NB, D), lambda i: (i, 0)),
        out_shape=jax.ShapeDtypeStruct((NPAD, D), jnp.float32),
    )(s1, tab, b1, W2)


def _tc_pool_body(s_ref, tab_ref, b_ref, batch_ref, out_ref, sums, cnts):
    i = pl.program_id(0)

    @pl.when(i == 0)
    def _():
        sums[...] = jnp.zeros_like(sums)
        cnts[...] = jnp.zeros_like(cnts)

    dinv = _dinv_from_tab(tab_ref[...])
    emb = (s_ref[0] + s_ref[1]) * dinv + b_ref[...]
    ids = batch_ref[0, 0, :]
    gids = lax.broadcasted_iota(jnp.int32, (NB, G), 1)
    p = (ids[:, None] == gids).astype(jnp.float32)
    sums[...] += lax.dot_general(p, emb, (((0,), (0,)), ((), ())),
                                 preferred_element_type=jnp.float32)
    cnts[...] += lax.dot_general(p, jnp.ones_like(emb),
                                 (((0,), (0,)), ((), ())),
                                 preferred_element_type=jnp.float32)

    @pl.when(i == NBLK - 1)
    def _():
        out_ref[...] = sums[...] / jnp.maximum(cnts[...], 1.0)


def _tc_pool(s2, tab, b2, batch3):
    return pl.pallas_call(
        _tc_pool_body,
        grid=(NBLK,),
        in_specs=[
            pl.BlockSpec((NC, NB, D), lambda i: (0, i, 0)),
            pl.BlockSpec((NC, NB, 16), lambda i: (0, i, 0)),
            pl.BlockSpec((1, D), lambda i: (0, 0)),
            pl.BlockSpec((1, 1, NB), lambda i: (i, 0, 0)),
        ],
        out_specs=pl.BlockSpec((G, D), lambda i: (0, 0)),
        out_shape=jax.ShapeDtypeStruct((G, D), jnp.float32),
        scratch_shapes=[
            pltpu.VMEM((G, D), jnp.float32),
            pltpu.VMEM((G, D), jnp.float32),
        ],
    )(s2, tab, b2, batch3)


def kernel(x, edge_index, batch, W1, b1, W2, b2):
    src = edge_index[0]
    dst = edge_index[1]
    epad = E_PAD - E
    src_p = jnp.concatenate(
        [src, jnp.zeros((epad,), jnp.int32)]).reshape(NC, NS, NCH, CH)
    dst_p = jnp.concatenate(
        [dst, jnp.full((epad,), N, jnp.int32)]).reshape(NC, NS, NCH, CH)
    x_pad = jnp.concatenate(
        [x, jnp.zeros((NPAD - N, D), jnp.float32)], axis=0)
    batch_p = jnp.concatenate(
        [batch, jnp.full((NPAD - N,), G, jnp.int32)]).reshape(NBLK, 1, NB)
    ones16 = jnp.ones((CH, 16), jnp.float32)
    zeros16 = jnp.zeros((RPT, 16), jnp.float32)
    zeros = jnp.zeros((RPT, D), jnp.float32)

    tab = _sc_degree(dst_p, ones16, zeros16)
    h1p = _tc_pre(x_pad, W1, tab)
    s1 = _sc_scatter(h1p, src_p, dst_p, zeros)
    h2p = _tc_mid(s1, tab, b1.reshape(1, D), W2)
    s2 = _sc_scatter(h2p, src_p, dst_p, zeros)
    return _tc_pool(s2, tab, b2.reshape(1, D), batch_p)


# speedup vs baseline: 22.3411x; 2.2551x over previous
import functools

import jax
import jax.numpy as jnp
from jax import lax
from jax.experimental import pallas as pl
from jax.experimental.pallas import tpu as pltpu
from jax.experimental.pallas import tpu_sc as plsc

N = 10000
E = 320000
D = 128
G = 128

NC = 2
NS = 16
NPAD = 10240
RPT = NPAD // NS
EPT = 10240
CH = 128
NCH = EPT // CH
E_PAD = NC * NS * EPT

NB = 2048
NBLK = NPAD // NB

_mesh = plsc.VectorSubcoreMesh(
    core_axis_name="c", subcore_axis_name="s", num_cores=NC, num_subcores=NS)


def _make_sc_degree():
    @functools.partial(
        pl.kernel,
        out_type=jax.ShapeDtypeStruct((NC, NPAD, 16), jnp.float32),
        mesh=_mesh,
        scratch_types=[
            pltpu.VMEM((NCH, CH), jnp.int32),
            pltpu.VMEM((CH, 16), jnp.float32),
            pltpu.VMEM_SHARED((NPAD, 16), jnp.float32),
        ],
    )
    def deg_kernel(dst_hbm, ones_hbm, zeros16_hbm, out_hbm, idx_v, ones_v,
                   acc_sh):
        cid = lax.axis_index("c")
        sid = lax.axis_index("s")
        pltpu.sync_copy(zeros16_hbm, acc_sh.at[pl.ds(sid * RPT, RPT)])
        pltpu.sync_copy(ones_hbm, ones_v)
        pltpu.sync_copy(dst_hbm.at[cid, sid], idx_v)
        plsc.subcore_barrier()

        def chunk(c, carry):
            pltpu.sync_copy(ones_v, acc_sh.at[idx_v.at[c]], add=True)
            return carry

        lax.fori_loop(0, NCH, chunk, 0)
        plsc.subcore_barrier()
        pltpu.sync_copy(acc_sh.at[pl.ds(sid * RPT, RPT)],
                        out_hbm.at[cid, pl.ds(sid * RPT, RPT)])

    return deg_kernel


_sc_degree = _make_sc_degree()


def _make_sc_scatter():
    @functools.partial(
        pl.kernel,
        out_type=jax.ShapeDtypeStruct((NC, NPAD, D), jnp.float32),
        mesh=_mesh,
        scratch_types=[
            pltpu.VMEM((NCH, CH), jnp.int32),
            pltpu.VMEM((NCH, CH), jnp.int32),
            pltpu.VMEM((CH, D), jnp.float32),
            pltpu.VMEM_SHARED((NPAD, D), jnp.float32),
            pltpu.SemaphoreType.DMA,
        ],
    )
    def scatter_kernel(hp_hbm, src_hbm, dst_hbm, zeros_hbm, out_hbm,
                       src_v, dst_v, rows_v, acc_sh, sem):
        cid = lax.axis_index("c")
        sid = lax.axis_index("s")
        row0 = sid * RPT

        @pl.when(cid == 0)
        def _():
            pltpu.sync_copy(hp_hbm.at[pl.ds(row0, RPT)],
                            acc_sh.at[pl.ds(row0, RPT)])

        @pl.when(cid != 0)
        def _():
            pltpu.sync_copy(zeros_hbm, acc_sh.at[pl.ds(row0, RPT)])

        pltpu.sync_copy(src_hbm.at[cid, sid], src_v)
        pltpu.sync_copy(dst_hbm.at[cid, sid], dst_v)
        plsc.subcore_barrier()

        def chunk(c, carry):
            pltpu.async_copy(hp_hbm.at[src_v.at[c]], rows_v, sem).wait()
            pltpu.sync_copy(rows_v, acc_sh.at[dst_v.at[c]], add=True)
            return carry

        lax.fori_loop(0, NCH, chunk, 0)
        plsc.subcore_barrier()
        pltpu.sync_copy(acc_sh.at[pl.ds(row0, RPT)],
                        out_hbm.at[cid, pl.ds(row0, RPT)])

    return scatter_kernel


_sc_scatter = _make_sc_scatter()


def _dinv_from_tab(tab_blk):
    deg = tab_blk[0, :, 0] + tab_blk[1, :, 0] + 1.0
    return lax.rsqrt(deg)[:, None]


def _tc_pre_body(x_ref, w_ref, tab_ref, out_ref):
    h = jnp.dot(x_ref[...], w_ref[...], preferred_element_type=jnp.float32)
    out_ref[...] = h * _dinv_from_tab(tab_ref[...])


def _tc_pre(x_pad, W1, tab):
    return pl.pallas_call(
        _tc_pre_body,
        grid=(NBLK,),
        in_specs=[
            pl.BlockSpec((NB, D), lambda i: (i, 0)),
            pl.BlockSpec((D, D), lambda i: (0, 0)),
            pl.BlockSpec((NC, NB, 16), lambda i: (0, i, 0)),
        ],
        out_specs=pl.BlockSpec((NB, D), lambda i: (i, 0)),
        out_shape=jax.ShapeDtypeStruct((NPAD, D), jnp.float32),
    )(x_pad, W1, tab)


def _tc_mid_body(s_ref, tab_ref, b_ref, w_ref, out_ref):
    dinv = _dinv_from_tab(tab_ref[...])
    s = s_ref[0] + s_ref[1]
    z = jnp.maximum(s * dinv + b_ref[...], 0.0)
    h = jnp.dot(z, w_ref[...], preferred_element_type=jnp.float32)
    out_ref[...] = h * dinv


def _tc_mid(s1, tab, b1, W2):
    return pl.pallas_call(
        _tc_mid_body,
        grid=(NBLK,),
        in_specs=[
            pl.BlockSpec((NC, NB, D), lambda i: (0, i, 0)),
            pl.BlockSpec((NC, NB, 16), lambda i: (0, i, 0)),
            pl.BlockSpec((1, D), lambda i: (0, 0)),
            pl.BlockSpec((D, D), lambda i: (0, 0)),
        ],
        out_specs=pl.BlockSpec((NB, D), lambda i: (i, 0)),
        out_shape=jax.ShapeDtypeStruct((NPAD, D), jnp.float32),
    )(s1, tab, b1, W2)


def _tc_pool_body(s_ref, tab_ref, b_ref, batch_ref, out_ref, sums, cnts):
    i = pl.program_id(0)

    @pl.when(i == 0)
    def _():
        sums[...] = jnp.zeros_like(sums)
        cnts[...] = jnp.zeros_like(cnts)

    dinv = _dinv_from_tab(tab_ref[...])
    emb = (s_ref[0] + s_ref[1]) * dinv + b_ref[...]
    ids = batch_ref[0, 0, :]
    gids = lax.broadcasted_iota(jnp.int32, (NB, G), 1)
    p = (ids[:, None] == gids).astype(jnp.float32)
    sums[...] += lax.dot_general(p, emb, (((0,), (0,)), ((), ())),
                                 preferred_element_type=jnp.float32)
    cnts[...] += lax.dot_general(p, jnp.ones_like(emb),
                                 (((0,), (0,)), ((), ())),
                                 preferred_element_type=jnp.float32)

    @pl.when(i == NBLK - 1)
    def _():
        out_ref[...] = sums[...] / jnp.maximum(cnts[...], 1.0)


def _tc_pool(s2, tab, b2, batch3):
    return pl.pallas_call(
        _tc_pool_body,
        grid=(NBLK,),
        in_specs=[
            pl.BlockSpec((NC, NB, D), lambda i: (0, i, 0)),
            pl.BlockSpec((NC, NB, 16), lambda i: (0, i, 0)),
            pl.BlockSpec((1, D), lambda i: (0, 0)),
            pl.BlockSpec((1, 1, NB), lambda i: (i, 0, 0)),
        ],
        out_specs=pl.BlockSpec((G, D), lambda i: (0, 0)),
        out_shape=jax.ShapeDtypeStruct((G, D), jnp.float32),
        scratch_shapes=[
            pltpu.VMEM((G, D), jnp.float32),
            pltpu.VMEM((G, D), jnp.float32),
        ],
    )(s2, tab, b2, batch3)


def kernel(x, edge_index, batch, W1, b1, W2, b2):
    src = edge_index[0]
    dst = edge_index[1]
    epad = E_PAD - E
    pad_ids = jnp.arange(epad, dtype=jnp.int32)
    src_p = jnp.concatenate(
        [src, pad_ids % N]).reshape(NC, NS, NCH, CH)
    dst_p = jnp.concatenate(
        [dst, N + pad_ids % (NPAD - N)]).reshape(NC, NS, NCH, CH)
    x_pad = jnp.concatenate(
        [x, jnp.zeros((NPAD - N, D), jnp.float32)], axis=0)
    batch_p = jnp.concatenate(
        [batch, jnp.full((NPAD - N,), G, jnp.int32)]).reshape(NBLK, 1, NB)
    ones16 = jnp.ones((CH, 16), jnp.float32)
    zeros16 = jnp.zeros((RPT, 16), jnp.float32)
    zeros = jnp.zeros((RPT, D), jnp.float32)

    tab = _sc_degree(dst_p, ones16, zeros16)
    h1p = _tc_pre(x_pad, W1, tab)
    s1 = _sc_scatter(h1p, src_p, dst_p, zeros)
    h2p = _tc_mid(s1, tab, b1.reshape(1, D), W2)
    s2 = _sc_scatter(h2p, src_p, dst_p, zeros)
    return _tc_pool(s2, tab, b2.reshape(1, D), batch_p)


# speedup vs baseline: 23.5906x; 1.0559x over previous
import functools

import jax
import jax.numpy as jnp
from jax import lax
from jax.experimental import pallas as pl
from jax.experimental.pallas import tpu as pltpu
from jax.experimental.pallas import tpu_sc as plsc

N = 10000
E = 320000
D = 128
G = 128

NC = 2
NS = 16
NPAD = 10240
RPT = NPAD // NS
EPT = 10240
CH = 128
NCH = EPT // CH
E_PAD = NC * NS * EPT

NB = 2048
NBLK = NPAD // NB

_mesh = plsc.VectorSubcoreMesh(
    core_axis_name="c", subcore_axis_name="s", num_cores=NC, num_subcores=NS)


NW = NC * NS
NROW = NPAD // D


def _make_sc_degree():
    @functools.partial(
        pl.kernel,
        out_type=jax.ShapeDtypeStruct((NW, NROW, D), jnp.float32),
        mesh=_mesh,
        compiler_params=pltpu.CompilerParams(needs_layout_passes=False),
        scratch_types=[
            pltpu.VMEM((NCH, CH), jnp.int32),
            pltpu.VMEM((NROW, D), jnp.float32),
        ],
    )
    def deg_kernel(dst_hbm, out_hbm, idx_v, acc):
        cid = lax.axis_index("c")
        sid = lax.axis_index("s")
        wid = cid * NS + sid
        pltpu.sync_copy(dst_hbm.at[cid, sid], idx_v)

        zero = jnp.zeros((16,), jnp.float32)

        def zrow(q, carry):
            for k in range(8):
                acc[q, pl.ds(k * 16, 16)] = zero
            return carry

        lax.fori_loop(0, NROW, zrow, 0)

        def hist(r, carry):
            for k in range(8):
                v = idx_v[r, pl.ds(k * 16, 16)]
                cnt, last = plsc.scan_count(v)
                plsc.addupdate_scatter(
                    acc,
                    [v >> 7, v & 127],
                    cnt.astype(jnp.float32),
                    mask=last,
                )
            return carry

        lax.fori_loop(0, NCH, hist, 0)
        pltpu.sync_copy(acc, out_hbm.at[wid])

    return deg_kernel


_sc_degree = _make_sc_degree()


def _make_sc_scatter():
    @functools.partial(
        pl.kernel,
        out_type=jax.ShapeDtypeStruct((NC, NPAD, D), jnp.float32),
        mesh=_mesh,
        scratch_types=[
            pltpu.VMEM((NCH, CH), jnp.int32),
            pltpu.VMEM((NCH, CH), jnp.int32),
            pltpu.VMEM((CH, D), jnp.float32),
            pltpu.VMEM_SHARED((NPAD, D), jnp.float32),
            pltpu.SemaphoreType.DMA,
        ],
    )
    def scatter_kernel(hp_hbm, src_hbm, dst_hbm, out_hbm,
                       src_v, dst_v, rows_v, acc_sh, sem):
        cid = lax.axis_index("c")
        sid = lax.axis_index("s")
        row0 = sid * RPT

        @pl.when(cid == 0)
        def _():
            pltpu.sync_copy(hp_hbm.at[pl.ds(row0, RPT)],
                            acc_sh.at[pl.ds(row0, RPT)])

        @pl.when(cid != 0)
        def _():
            def zrow(r, carry):
                for k in range(8):
                    rows_v[r, pl.ds(k * 16, 16)] = jnp.zeros((16,),
                                                             jnp.float32)
                return carry
            lax.fori_loop(0, CH, zrow, 0)
            for b in range(RPT // CH):
                pltpu.sync_copy(rows_v,
                                acc_sh.at[pl.ds(row0 + b * CH, CH)])

        pltpu.sync_copy(src_hbm.at[cid, sid], src_v)
        pltpu.sync_copy(dst_hbm.at[cid, sid], dst_v)
        plsc.subcore_barrier()

        def chunk(c, carry):
            pltpu.async_copy(hp_hbm.at[src_v.at[c]], rows_v, sem).wait()
            pltpu.sync_copy(rows_v, acc_sh.at[dst_v.at[c]], add=True)
            return carry

        lax.fori_loop(0, NCH, chunk, 0)
        plsc.subcore_barrier()
        pltpu.sync_copy(acc_sh.at[pl.ds(row0, RPT)],
                        out_hbm.at[cid, pl.ds(row0, RPT)])

    return scatter_kernel


_sc_scatter = _make_sc_scatter()


def _dinv_from_tab(tab_blk):
    nq = NB // D
    t = jnp.sum(tab_blk, axis=0)
    n_idx = lax.broadcasted_iota(jnp.int32, (NB, nq), 0)
    q_idx = lax.broadcasted_iota(jnp.int32, (NB, nq), 1)
    a = (n_idx // D == q_idx).astype(jnp.float32)
    r = jnp.dot(a, t, preferred_element_type=jnp.float32)
    l_idx = lax.broadcasted_iota(jnp.int32, (NB, D), 1)
    m_idx = lax.broadcasted_iota(jnp.int32, (NB, D), 0)
    m = (l_idx == m_idx % D).astype(jnp.float32)
    deg = jnp.sum(r * m, axis=1, keepdims=True) + 1.0
    return lax.rsqrt(deg)


def _tc_pre_body(x_ref, w_ref, tab_ref, out_ref):
    h = jnp.dot(x_ref[...], w_ref[...], preferred_element_type=jnp.float32)
    out_ref[...] = h * _dinv_from_tab(tab_ref[...])


def _tc_pre(x_pad, W1, tab):
    return pl.pallas_call(
        _tc_pre_body,
        grid=(NBLK,),
        in_specs=[
            pl.BlockSpec((NB, D), lambda i: (i, 0)),
            pl.BlockSpec((D, D), lambda i: (0, 0)),
            pl.BlockSpec((NW, NB // D, D), lambda i: (0, i, 0)),
        ],
        out_specs=pl.BlockSpec((NB, D), lambda i: (i, 0)),
        out_shape=jax.ShapeDtypeStruct((NPAD, D), jnp.float32),
    )(x_pad, W1, tab)


def _tc_mid_body(s_ref, tab_ref, b_ref, w_ref, out_ref):
    dinv = _dinv_from_tab(tab_ref[...])
    s = s_ref[0] + s_ref[1]
    z = jnp.maximum(s * dinv + b_ref[...], 0.0)
    h = jnp.dot(z, w_ref[...], preferred_element_type=jnp.float32)
    out_ref[...] = h * dinv


def _tc_mid(s1, tab, b1, W2):
    return pl.pallas_call(
        _tc_mid_body,
        grid=(NBLK,),
        in_specs=[
            pl.BlockSpec((NC, NB, D), lambda i: (0, i, 0)),
            pl.BlockSpec((NW, NB // D, D), lambda i: (0, i, 0)),
            pl.BlockSpec((1, D), lambda i: (0, 0)),
            pl.BlockSpec((D, D), lambda i: (0, 0)),
        ],
        out_specs=pl.BlockSpec((NB, D), lambda i: (i, 0)),
        out_shape=jax.ShapeDtypeStruct((NPAD, D), jnp.float32),
    )(s1, tab, b1, W2)


def _tc_pool_body(s_ref, tab_ref, b_ref, batch_ref, out_ref, sums, cnts):
    i = pl.program_id(0)

    @pl.when(i == 0)
    def _():
        sums[...] = jnp.zeros_like(sums)
        cnts[...] = jnp.zeros_like(cnts)

    dinv = _dinv_from_tab(tab_ref[...])
    emb = (s_ref[0] + s_ref[1]) * dinv + b_ref[...]
    ids = batch_ref[0, 0, :]
    gids = lax.broadcasted_iota(jnp.int32, (NB, G), 1)
    p = (ids[:, None] == gids).astype(jnp.float32)
    sums[...] += lax.dot_general(p, emb, (((0,), (0,)), ((), ())),
                                 preferred_element_type=jnp.float32)
    cnts[...] += lax.dot_general(p, jnp.ones_like(emb),
                                 (((0,), (0,)), ((), ())),
                                 preferred_element_type=jnp.float32)

    @pl.when(i == NBLK - 1)
    def _():
        out_ref[...] = sums[...] / jnp.maximum(cnts[...], 1.0)


def _tc_pool(s2, tab, b2, batch3):
    return pl.pallas_call(
        _tc_pool_body,
        grid=(NBLK,),
        in_specs=[
            pl.BlockSpec((NC, NB, D), lambda i: (0, i, 0)),
            pl.BlockSpec((NW, NB // D, D), lambda i: (0, i, 0)),
            pl.BlockSpec((1, D), lambda i: (0, 0)),
            pl.BlockSpec((1, 1, NB), lambda i: (i, 0, 0)),
        ],
        out_specs=pl.BlockSpec((G, D), lambda i: (0, 0)),
        out_shape=jax.ShapeDtypeStruct((G, D), jnp.float32),
        scratch_shapes=[
            pltpu.VMEM((G, D), jnp.float32),
            pltpu.VMEM((G, D), jnp.float32),
        ],
    )(s2, tab, b2, batch3)


def kernel(x, edge_index, batch, W1, b1, W2, b2):
    src = edge_index[0]
    dst = edge_index[1]
    epad = E_PAD - E
    pad_ids = jnp.arange(epad, dtype=jnp.int32)
    src_p = jnp.concatenate(
        [src, pad_ids % N]).reshape(NC, NS, NCH, CH)
    dst_p = jnp.concatenate(
        [dst, N + pad_ids % (NPAD - N)]).reshape(NC, NS, NCH, CH)
    x_pad = jnp.concatenate(
        [x, jnp.zeros((NPAD - N, D), jnp.float32)], axis=0)
    batch_p = jnp.concatenate(
        [batch, jnp.full((NPAD - N,), G, jnp.int32)]).reshape(NBLK, 1, NB)

    tab = _sc_degree(dst_p)
    h1p = _tc_pre(x_pad, W1, tab)
    s1 = _sc_scatter(h1p, src_p, dst_p)
    h2p = _tc_mid(s1, tab, b1.reshape(1, D), W2)
    s2 = _sc_scatter(h2p, src_p, dst_p)
    return _tc_pool(s2, tab, b2.reshape(1, D), batch_p)


# speedup vs baseline: 29.7756x; 1.2622x over previous
import functools

import jax
import jax.numpy as jnp
from jax import lax
from jax.experimental import pallas as pl
from jax.experimental.pallas import tpu as pltpu
from jax.experimental.pallas import tpu_sc as plsc

N = 10000
E = 320000
D = 128
G = 128

NC = 2
NS = 16
NPAD = 10240
RPT = NPAD // NS
EPT = 10240
CH = 128
NCH = EPT // CH
E_PAD = NC * NS * EPT

NB = 2048
NBLK = NPAD // NB

_mesh = plsc.VectorSubcoreMesh(
    core_axis_name="c", subcore_axis_name="s", num_cores=NC, num_subcores=NS)


NW = NC * NS
NROW = NPAD // D


def _make_sc_degree():
    @functools.partial(
        pl.kernel,
        out_type=jax.ShapeDtypeStruct((NW, NROW, D), jnp.float32),
        mesh=_mesh,
        compiler_params=pltpu.CompilerParams(needs_layout_passes=False),
        scratch_types=[
            pltpu.VMEM((NCH, CH), jnp.int32),
            pltpu.VMEM((NROW, D), jnp.float32),
        ],
    )
    def deg_kernel(dst_hbm, out_hbm, idx_v, acc):
        cid = lax.axis_index("c")
        sid = lax.axis_index("s")
        wid = cid * NS + sid
        pltpu.sync_copy(dst_hbm.at[cid, sid], idx_v)

        zero = jnp.zeros((16,), jnp.float32)

        def zrow(q, carry):
            for k in range(8):
                acc[q, pl.ds(k * 16, 16)] = zero
            return carry

        lax.fori_loop(0, NROW, zrow, 0)

        def hist(r, carry):
            for k in range(8):
                v = idx_v[r, pl.ds(k * 16, 16)]
                cnt, last = plsc.scan_count(v)
                plsc.addupdate_scatter(
                    acc,
                    [v >> 7, v & 127],
                    cnt.astype(jnp.float32),
                    mask=last,
                )
            return carry

        lax.fori_loop(0, NCH, hist, 0)
        pltpu.sync_copy(acc, out_hbm.at[wid])

    return deg_kernel


_sc_degree = _make_sc_degree()


def _make_sc_scatter():
    @functools.partial(
        pl.kernel,
        out_type=jax.ShapeDtypeStruct((NC, NPAD, D), jnp.float32),
        mesh=_mesh,
        scratch_types=[
            pltpu.VMEM((NCH // 2, CH), jnp.int32),
            pltpu.VMEM((NCH // 2, CH), jnp.int32),
            pltpu.VMEM((2, CH, D), jnp.float32),
            pltpu.VMEM_SHARED((NPAD, D), jnp.float32),
            pltpu.SemaphoreType.DMA,
            pltpu.SemaphoreType.DMA,
        ],
    )
    def scatter_kernel(hp_hbm, src_hbm, dst_hbm, out_hbm,
                       src_v, dst_v, rows_v, acc_sh, sem0, sem1):
        cid = lax.axis_index("c")
        sid = lax.axis_index("s")
        row0 = sid * RPT

        @pl.when(cid == 0)
        def _():
            pltpu.sync_copy(hp_hbm.at[pl.ds(row0, RPT)],
                            acc_sh.at[pl.ds(row0, RPT)])

        @pl.when(cid != 0)
        def _():
            def zrow(r, carry):
                for k in range(8):
                    rows_v[0, r, pl.ds(k * 16, 16)] = jnp.zeros((16,),
                                                                jnp.float32)
                return carry
            lax.fori_loop(0, CH, zrow, 0)
            for b in range(RPT // CH):
                pltpu.sync_copy(rows_v.at[0],
                                acc_sh.at[pl.ds(row0 + b * CH, CH)])

        plsc.subcore_barrier()

        def gather(c, buf, sem):
            return pltpu.make_async_copy(hp_hbm.at[src_v.at[c]],
                                         rows_v.at[buf], sem)

        nh = NCH // 2
        for h in range(2):
            pltpu.sync_copy(src_hbm.at[cid, sid, pl.ds(h * nh, nh)], src_v)
            pltpu.sync_copy(dst_hbm.at[cid, sid, pl.ds(h * nh, nh)], dst_v)
            gather(0, 0, sem0).start()

            def pair(p, carry):
                c0 = 2 * p
                c1 = c0 + 1
                gather(c0, 0, sem0).wait()
                gather(c1, 1, sem1).start()
                pltpu.sync_copy(rows_v.at[0], acc_sh.at[dst_v.at[c0]],
                                add=True)
                gather(c1, 1, sem1).wait()

                @pl.when(c1 + 1 < nh)
                def _():
                    gather(c1 + 1, 0, sem0).start()

                pltpu.sync_copy(rows_v.at[1], acc_sh.at[dst_v.at[c1]],
                                add=True)
                return carry

            lax.fori_loop(0, nh // 2, pair, 0)
        plsc.subcore_barrier()
        pltpu.sync_copy(acc_sh.at[pl.ds(row0, RPT)],
                        out_hbm.at[cid, pl.ds(row0, RPT)])

    return scatter_kernel


_sc_scatter = _make_sc_scatter()


def _dinv_from_tab(tab_blk):
    nq = NB // D
    t = jnp.sum(tab_blk, axis=0)
    n_idx = lax.broadcasted_iota(jnp.int32, (NB, nq), 0)
    q_idx = lax.broadcasted_iota(jnp.int32, (NB, nq), 1)
    a = (n_idx // D == q_idx).astype(jnp.float32)
    r = jnp.dot(a, t, preferred_element_type=jnp.float32)
    l_idx = lax.broadcasted_iota(jnp.int32, (NB, D), 1)
    m_idx = lax.broadcasted_iota(jnp.int32, (NB, D), 0)
    m = (l_idx == m_idx % D).astype(jnp.float32)
    deg = jnp.sum(r * m, axis=1, keepdims=True) + 1.0
    return lax.rsqrt(deg)


def _tc_pre_body(x_ref, w_ref, tab_ref, out_ref):
    h = jnp.dot(x_ref[...], w_ref[...], preferred_element_type=jnp.float32)
    out_ref[...] = h * _dinv_from_tab(tab_ref[...])


def _tc_pre(x_pad, W1, tab):
    return pl.pallas_call(
        _tc_pre_body,
        grid=(NBLK,),
        in_specs=[
            pl.BlockSpec((NB, D), lambda i: (i, 0)),
            pl.BlockSpec((D, D), lambda i: (0, 0)),
            pl.BlockSpec((NW, NB // D, D), lambda i: (0, i, 0)),
        ],
        out_specs=pl.BlockSpec((NB, D), lambda i: (i, 0)),
        out_shape=jax.ShapeDtypeStruct((NPAD, D), jnp.float32),
    )(x_pad, W1, tab)


def _tc_mid_body(s_ref, tab_ref, b_ref, w_ref, out_ref):
    dinv = _dinv_from_tab(tab_ref[...])
    s = s_ref[0] + s_ref[1]
    z = jnp.maximum(s * dinv + b_ref[...], 0.0)
    h = jnp.dot(z, w_ref[...], preferred_element_type=jnp.float32)
    out_ref[...] = h * dinv


def _tc_mid(s1, tab, b1, W2):
    return pl.pallas_call(
        _tc_mid_body,
        grid=(NBLK,),
        in_specs=[
            pl.BlockSpec((NC, NB, D), lambda i: (0, i, 0)),
            pl.BlockSpec((NW, NB // D, D), lambda i: (0, i, 0)),
            pl.BlockSpec((1, D), lambda i: (0, 0)),
            pl.BlockSpec((D, D), lambda i: (0, 0)),
        ],
        out_specs=pl.BlockSpec((NB, D), lambda i: (i, 0)),
        out_shape=jax.ShapeDtypeStruct((NPAD, D), jnp.float32),
    )(s1, tab, b1, W2)


def _tc_pool_body(s_ref, tab_ref, b_ref, batch_ref, out_ref, sums, cnts):
    i = pl.program_id(0)

    @pl.when(i == 0)
    def _():
        sums[...] = jnp.zeros_like(sums)
        cnts[...] = jnp.zeros_like(cnts)

    dinv = _dinv_from_tab(tab_ref[...])
    emb = (s_ref[0] + s_ref[1]) * dinv + b_ref[...]
    ids = batch_ref[0, 0, :]
    gids = lax.broadcasted_iota(jnp.int32, (NB, G), 1)
    p = (ids[:, None] == gids).astype(jnp.float32)
    sums[...] += lax.dot_general(p, emb, (((0,), (0,)), ((), ())),
                                 preferred_element_type=jnp.float32)
    cnts[...] += lax.dot_general(p, jnp.ones_like(emb),
                                 (((0,), (0,)), ((), ())),
                                 preferred_element_type=jnp.float32)

    @pl.when(i == NBLK - 1)
    def _():
        out_ref[...] = sums[...] / jnp.maximum(cnts[...], 1.0)


def _tc_pool(s2, tab, b2, batch3):
    return pl.pallas_call(
        _tc_pool_body,
        grid=(NBLK,),
        in_specs=[
            pl.BlockSpec((NC, NB, D), lambda i: (0, i, 0)),
            pl.BlockSpec((NW, NB // D, D), lambda i: (0, i, 0)),
            pl.BlockSpec((1, D), lambda i: (0, 0)),
            pl.BlockSpec((1, 1, NB), lambda i: (i, 0, 0)),
        ],
        out_specs=pl.BlockSpec((G, D), lambda i: (0, 0)),
        out_shape=jax.ShapeDtypeStruct((G, D), jnp.float32),
        scratch_shapes=[
            pltpu.VMEM((G, D), jnp.float32),
            pltpu.VMEM((G, D), jnp.float32),
        ],
    )(s2, tab, b2, batch3)


def kernel(x, edge_index, batch, W1, b1, W2, b2):
    src = edge_index[0]
    dst = edge_index[1]
    epad = E_PAD - E
    pad_ids = jnp.arange(epad, dtype=jnp.int32)
    src_p = jnp.concatenate(
        [src, pad_ids % N]).reshape(NC, NS, NCH, CH)
    dst_p = jnp.concatenate(
        [dst, N + pad_ids % (NPAD - N)]).reshape(NC, NS, NCH, CH)
    x_pad = jnp.concatenate(
        [x, jnp.zeros((NPAD - N, D), jnp.float32)], axis=0)
    batch_p = jnp.concatenate(
        [batch, jnp.full((NPAD - N,), G, jnp.int32)]).reshape(NBLK, 1, NB)

    tab = _sc_degree(dst_p)
    h1p = _tc_pre(x_pad, W1, tab)
    s1 = _sc_scatter(h1p, src_p, dst_p)
    h2p = _tc_mid(s1, tab, b1.reshape(1, D), W2)
    s2 = _sc_scatter(h2p, src_p, dst_p)
    return _tc_pool(s2, tab, b2.reshape(1, D), batch_p)
